# Initial kernel scaffold; baseline (speedup 1.0000x reference)
import functools

import jax
import jax.numpy as jnp
from jax import lax
from jax.experimental import pallas as pl
from jax.experimental.pallas import tpu as pltpu
from jax.experimental.pallas import tpu_sc as plsc

N = 10000
NPAD = 10240
E = 640000
K = 160
EPAD = 32 * K * 128
F32 = jnp.float32
I32 = jnp.int32


def _tc_a_body(x_ref, w1p_ref, c1_ref, as8_ref, hx_ref, att_ref):
    hx = jnp.dot(x_ref[...], w1p_ref[...], preferred_element_type=F32) + c1_ref[...]
    hx_ref[...] = hx
    a8 = jnp.dot(hx, as8_ref[...], preferred_element_type=F32)
    rid = lax.broadcasted_iota(I32, (NPAD, 8), 0)
    att_ref[...] = jnp.where(rid < N, a8, -1e30)


def _tc_b_body(p0_ref, p1_ref, rp1_ref, b1_ref, w2p_ref, c2_ref, as2_ref,
               hx_ref, att_ref):
    acc = p0_ref[...] + p1_ref[...]
    num = acc[:, :32]
    denb = jnp.dot(acc, rp1_ref[...], preferred_element_type=F32)
    out1 = num / (denb + 1e-16) + b1_ref[...]
    x2 = jnp.where(out1 > 0, out1, jnp.exp(out1) - 1.0)
    hx2 = jnp.dot(x2, w2p_ref[...], preferred_element_type=F32) + c2_ref[...]
    hx_ref[...] = hx2
    a2 = jnp.dot(hx2, as2_ref[...], preferred_element_type=F32)
    rid = lax.broadcasted_iota(I32, (NPAD, 2), 0)
    att_ref[...] = jnp.where(rid < N, a2, -1e30)


def _tc_c_body(p0_ref, p1_ref, rp2_ref, b2_ref, out_ref):
    acc = p0_ref[...] + p1_ref[...]
    num = acc[:, :64]
    denb = jnp.dot(acc, rp2_ref[...], preferred_element_type=F32)
    out_ref[...] = num / (denb + 1e-16) + b2_ref[...]


_tc_a = pl.pallas_call(
    _tc_a_body,
    out_shape=[jax.ShapeDtypeStruct((NPAD, 48), F32),
               jax.ShapeDtypeStruct((NPAD, 8), F32)],
)

_tc_b = pl.pallas_call(
    _tc_b_body,
    out_shape=[jax.ShapeDtypeStruct((NPAD, 80), F32),
               jax.ShapeDtypeStruct((NPAD, 2), F32)],
)

_tc_c = pl.pallas_call(
    _tc_c_body,
    out_shape=jax.ShapeDtypeStruct((NPAD, 64), F32),
)


def _make_sc(rw, heads):
    ac = 2 * heads
    nv = rw // 16

    mesh = plsc.VectorSubcoreMesh(core_axis_name="c", subcore_axis_name="s")

    @functools.partial(
        pl.kernel,
        out_type=jax.ShapeDtypeStruct((2, NPAD, rw), F32),
        mesh=mesh,
        scratch_types=[
            pltpu.VMEM((NPAD, ac), F32),
            pltpu.VMEM((2, 128), I32),
            pltpu.VMEM((128, rw), F32),
            pltpu.VMEM((heads + 1, 128), F32),
            pltpu.VMEM((64, rw), F32),
            pltpu.SemaphoreType.DMA,
            pltpu.VMEM_SHARED((NPAD, rw), F32),
        ],
    )
    def sc_kernel(idx_hbm, hx_hbm, att_hbm, out_hbm,
                  att_v, ibuf, hbuf, wbuf, zbuf, sem, accum):
        cid = lax.axis_index("c")
        sid = lax.axis_index("s")
        wid = sid * 2 + cid
        base = sid * (NPAD // 16)

        def zr(r, _):
            for cc in range(nv):
                zbuf[r, pl.ds(cc * 16, 16)] = jnp.zeros((16,), F32)
            return 0
        lax.fori_loop(0, 64, zr, 0)

        def zc(k, _):
            pltpu.sync_copy(zbuf, accum.at[pl.ds(base + k * 64, 64)])
            return 0
        lax.fori_loop(0, NPAD // 16 // 64, zc, 0)

        pltpu.sync_copy(att_hbm, att_v)
        for cc in range(8):
            wbuf[heads, pl.ds(cc * 16, 16)] = jnp.zeros((16,), F32)

        plsc.subcore_barrier()

        iota16 = lax.iota(I32, 16)
        if heads == 4:
            rowpats = [iota16 // 8, iota16 // 8 + 2, jnp.minimum(iota16, 4)]
        else:
            rowpats = [iota16 * 0 for _ in range(nv - 1)]
            rowpats.append(jnp.minimum(iota16, 1))

        def chunk(j, _):
            pltpu.sync_copy(idx_hbm.at[wid, j], ibuf)
            pltpu.async_copy(hx_hbm.at[ibuf.at[0]], hbuf, sem).wait()

            for g in range(8):
                s16 = ibuf[0, pl.ds(g * 16, 16)]
                d16 = ibuf[1, pl.ds(g * 16, 16)]
                for h in range(heads):
                    zs = plsc.load_gather(att_v, [s16, jnp.full((16,), h, I32)])
                    zd = plsc.load_gather(att_v, [d16, jnp.full((16,), heads + h, I32)])
                    z = zs + zd
                    wbuf[h, pl.ds(g * 16, 16)] = jnp.exp(jnp.maximum(z, 0.2 * z))

            def pedge(e, _):
                ecol = jnp.full((16,), 0, I32) + e
                for v in range(nv):
                    wv = plsc.load_gather(wbuf, [rowpats[v], ecol])
                    hbuf[e, pl.ds(v * 16, 16)] = hbuf[e, pl.ds(v * 16, 16)] * wv
                return 0
            lax.fori_loop(0, 128, pedge, 0)

            pltpu.sync_copy(hbuf, accum.at[ibuf.at[1]], add=True)
            return 0
        lax.fori_loop(0, K, chunk, 0)

        plsc.subcore_barrier()

        def co(k, _):
            pltpu.sync_copy(accum.at[pl.ds(base + k * 64, 64)], zbuf)
            pltpu.sync_copy(zbuf, out_hbm.at[cid, pl.ds(base + k * 64, 64)])
            return 0
        lax.fori_loop(0, NPAD // 16 // 64, co, 0)

    return sc_kernel


_sc1 = _make_sc(48, 4)
_sc2 = _make_sc(80, 1)


def kernel(x, edge_index, W1, att_src1, att_dst1, b1, W2, att_src2, att_dst2, b2):
    src = edge_index[0].astype(I32)
    dst = edge_index[1].astype(I32)
    loop = jnp.arange(N, dtype=I32)
    padn = EPAD - (E + N)
    src_p = jnp.concatenate([src, loop, jnp.zeros((padn,), I32)])
    dst_p = jnp.concatenate([dst, loop, N + (jnp.arange(padn, dtype=I32) % 240)])
    idx = jnp.stack([src_p, dst_p]).reshape(2, 32, K, 128).transpose(1, 2, 0, 3)

    x_pad = jnp.pad(x, ((0, NPAD - N), (0, 0)))

    m4 = jnp.repeat(jnp.eye(4, dtype=F32), 8, axis=0)
    as8 = jnp.concatenate([m4 * att_src1.reshape(32, 1),
                           m4 * att_dst1.reshape(32, 1)], axis=1)
    as8p = jnp.concatenate([as8, jnp.zeros((16, 8), F32)], axis=0)
    w1p = jnp.concatenate([W1, jnp.zeros((128, 16), F32)], axis=1)
    c1 = jnp.concatenate([jnp.zeros((32,), F32), jnp.ones((4,), F32),
                          jnp.zeros((12,), F32)]).reshape(1, 48)
    rp1 = jnp.zeros((48, 32), F32).at[32:36].set(m4.T)
    w2p = jnp.concatenate([W2, jnp.zeros((32, 16), F32)], axis=1)
    c2 = jnp.zeros((80,), F32).at[64].set(1.0).reshape(1, 80)
    as2p = jnp.zeros((80, 2), F32).at[:64].set(
        jnp.concatenate([att_src2.reshape(64, 1), att_dst2.reshape(64, 1)], axis=1))
    rp2 = jnp.zeros((80, 64), F32).at[64].set(jnp.ones((64,), F32))
    b1r = b1.reshape(1, 32)
    b2r = b2.reshape(1, 64)

    hx1, att1 = _tc_a(x_pad, w1p, c1, as8p)
    parts1 = _sc1(idx, hx1, att1)
    hx2, att2 = _tc_b(parts1[0], parts1[1], rp1, b1r, w2p, c2, as2p)
    parts2 = _sc2(idx, hx2, att2)
    out = _tc_c(parts2[0], parts2[1], rp2, b2r)
    return out[:N]


# speedup vs baseline: 40.1614x; 40.1614x over previous
import functools

import jax
import jax.numpy as jnp
from jax import lax
from jax.experimental import pallas as pl
from jax.experimental.pallas import tpu as pltpu
from jax.experimental.pallas import tpu_sc as plsc

N = 10000
NPAD = 10240
E = 640000
K = 160
EPAD = 32 * K * 128
F32 = jnp.float32
I32 = jnp.int32


def _tc_a_body(x_ref, w1p_ref, c1_ref, as8_ref, hx_ref, att_ref):
    hx = jnp.dot(x_ref[...], w1p_ref[...], preferred_element_type=F32) + c1_ref[...]
    hx_ref[...] = hx
    a8 = jnp.dot(hx, as8_ref[...], preferred_element_type=F32)
    rid = lax.broadcasted_iota(I32, (NPAD, 8), 0)
    att_ref[...] = jnp.where(rid < N, a8, -1e30)


def _tc_b_body(p0_ref, p1_ref, rp1_ref, b1_ref, w2p_ref, c2_ref, as2_ref,
               hx_ref, att_ref):
    acc = p0_ref[...] + p1_ref[...]
    num = acc[:, :32]
    denb = jnp.dot(acc, rp1_ref[...], preferred_element_type=F32)
    out1 = num / (denb + 1e-16) + b1_ref[...]
    x2 = jnp.where(out1 > 0, out1, jnp.exp(out1) - 1.0)
    hx2 = jnp.dot(x2, w2p_ref[...], preferred_element_type=F32) + c2_ref[...]
    hx_ref[...] = hx2
    a2 = jnp.dot(hx2, as2_ref[...], preferred_element_type=F32)
    rid = lax.broadcasted_iota(I32, (NPAD, 2), 0)
    att_ref[...] = jnp.where(rid < N, a2, -1e30)


def _tc_c_body(p0_ref, p1_ref, rp2_ref, b2_ref, out_ref):
    acc = p0_ref[...] + p1_ref[...]
    num = acc[:, :64]
    denb = jnp.dot(acc, rp2_ref[...], preferred_element_type=F32)
    out_ref[...] = num / (denb + 1e-16) + b2_ref[...]


_tc_a = pl.pallas_call(
    _tc_a_body,
    out_shape=[jax.ShapeDtypeStruct((NPAD, 48), F32),
               jax.ShapeDtypeStruct((NPAD, 8), F32)],
)

_tc_b = pl.pallas_call(
    _tc_b_body,
    out_shape=[jax.ShapeDtypeStruct((NPAD, 80), F32),
               jax.ShapeDtypeStruct((NPAD, 2), F32)],
)

_tc_c = pl.pallas_call(
    _tc_c_body,
    out_shape=jax.ShapeDtypeStruct((NPAD, 64), F32),
)


def _make_sc(rw, heads):
    ac = 2 * heads
    nv = rw // 16

    mesh = plsc.VectorSubcoreMesh(core_axis_name="c", subcore_axis_name="s",
                                  num_cores=2, num_subcores=16)

    @functools.partial(
        pl.kernel,
        out_type=jax.ShapeDtypeStruct((2, NPAD, rw), F32),
        mesh=mesh,
        compiler_params=pltpu.CompilerParams(needs_layout_passes=False,
                                             use_tc_tiling_on_sc=False),
        scratch_types=[
            pltpu.VMEM((NPAD * ac,), F32),
            pltpu.VMEM((2, 128), I32),
            pltpu.VMEM((128, rw), F32),
            pltpu.VMEM(((heads + 1) * 128,), F32),
            pltpu.VMEM((64, rw), F32),
            pltpu.SemaphoreType.DMA,
            pltpu.VMEM_SHARED((NPAD, rw), F32),
        ],
    )
    def sc_kernel(idx_hbm, hx_hbm, att_hbm, out_hbm,
                  att_v, ibuf, hbuf, wbuf, zbuf, sem, accum):
        cid = lax.axis_index("c")
        sid = lax.axis_index("s")
        wid = sid * 2 + cid
        base = sid * (NPAD // 16)

        def zr(r, _):
            for cc in range(nv):
                zbuf[r, pl.ds(cc * 16, 16)] = jnp.zeros((16,), F32)
            return 0
        lax.fori_loop(0, 64, zr, 0)

        def zc(k, _):
            pltpu.sync_copy(zbuf, accum.at[pl.ds(base + k * 64, 64)])
            return 0
        lax.fori_loop(0, NPAD // 16 // 64, zc, 0)

        pltpu.sync_copy(att_hbm, att_v)
        for cc in range(8):
            wbuf[pl.ds(heads * 128 + cc * 16, 16)] = jnp.zeros((16,), F32)

        plsc.subcore_barrier()

        def _rowpats():
            iota16 = lax.iota(I32, 16)
            if heads == 4:
                return [(iota16 // 8) * 128, (iota16 // 8) * 128 + 256,
                        jnp.minimum(iota16, 4) * 128]
            pats = [iota16 * 0 for _ in range(nv - 1)]
            pats.append(jnp.minimum(iota16, 1) * 128)
            return pats

        def chunk(j, _):
            pltpu.sync_copy(idx_hbm.at[wid, j], ibuf)
            pltpu.async_copy(hx_hbm.at[ibuf.at[0]], hbuf, sem).wait()

            for g in range(8):
                s16 = ibuf[0, pl.ds(g * 16, 16)]
                d16 = ibuf[1, pl.ds(g * 16, 16)]
                for h in range(heads):
                    zs = plsc.load_gather(att_v, [s16 * ac + h])
                    zd = plsc.load_gather(att_v, [d16 * ac + (heads + h)])
                    z = zs + zd
                    wbuf[pl.ds(h * 128 + g * 16, 16)] = jnp.exp(jnp.maximum(z, 0.2 * z))

            def pedge(e, _):
                rowpats = _rowpats()
                for v in range(nv):
                    wv = plsc.load_gather(wbuf, [rowpats[v] + e])
                    hbuf[e, pl.ds(v * 16, 16)] = hbuf[e, pl.ds(v * 16, 16)] * wv
                return 0
            lax.fori_loop(0, 128, pedge, 0)

            pltpu.sync_copy(hbuf, accum.at[ibuf.at[1]], add=True)
            return 0
        lax.fori_loop(0, K, chunk, 0)

        plsc.subcore_barrier()

        def co(k, _):
            pltpu.sync_copy(accum.at[pl.ds(base + k * 64, 64)], zbuf)
            pltpu.sync_copy(zbuf, out_hbm.at[cid, pl.ds(base + k * 64, 64)])
            return 0
        lax.fori_loop(0, NPAD // 16 // 64, co, 0)

    return sc_kernel


_sc1 = _make_sc(48, 4)
_sc2 = _make_sc(80, 1)


def kernel(x, edge_index, W1, att_src1, att_dst1, b1, W2, att_src2, att_dst2, b2):
    src = edge_index[0].astype(I32)
    dst = edge_index[1].astype(I32)
    loop = jnp.arange(N, dtype=I32)
    padn = EPAD - (E + N)
    src_p = jnp.concatenate([src, loop, jnp.zeros((padn,), I32)])
    dst_p = jnp.concatenate([dst, loop, N + (jnp.arange(padn, dtype=I32) % 240)])
    idx = jnp.stack([src_p, dst_p]).reshape(2, 32, K, 128).transpose(1, 2, 0, 3)

    x_pad = jnp.pad(x, ((0, NPAD - N), (0, 0)))

    m4 = jnp.repeat(jnp.eye(4, dtype=F32), 8, axis=0)
    as8 = jnp.concatenate([m4 * att_src1.reshape(32, 1),
                           m4 * att_dst1.reshape(32, 1)], axis=1)
    as8p = jnp.concatenate([as8, jnp.zeros((16, 8), F32)], axis=0)
    w1p = jnp.concatenate([W1, jnp.zeros((128, 16), F32)], axis=1)
    c1 = jnp.concatenate([jnp.zeros((32,), F32), jnp.ones((4,), F32),
                          jnp.zeros((12,), F32)]).reshape(1, 48)
    rp1 = jnp.zeros((48, 32), F32).at[32:36].set(m4.T)
    w2p = jnp.concatenate([W2, jnp.zeros((32, 16), F32)], axis=1)
    c2 = jnp.zeros((80,), F32).at[64].set(1.0).reshape(1, 80)
    as2p = jnp.zeros((80, 2), F32).at[:64].set(
        jnp.concatenate([att_src2.reshape(64, 1), att_dst2.reshape(64, 1)], axis=1))
    rp2 = jnp.zeros((80, 64), F32).at[64].set(jnp.ones((64,), F32))
    b1r = b1.reshape(1, 32)
    b2r = b2.reshape(1, 64)

    hx1, att1 = _tc_a(x_pad, w1p, c1, as8p)
    parts1 = _sc1(idx, hx1, att1.reshape(-1))
    hx2, att2 = _tc_b(parts1[0], parts1[1], rp1, b1r, w2p, c2, as2p)
    parts2 = _sc2(idx, hx2, att2.reshape(-1))
    out = _tc_c(parts2[0], parts2[1], rp2, b2r)
    return out[:N]


# speedup vs baseline: 50.6795x; 1.2619x over previous
import functools

import jax
import jax.numpy as jnp
from jax import lax
from jax.experimental import pallas as pl
from jax.experimental.pallas import tpu as pltpu
from jax.experimental.pallas import tpu_sc as plsc

N = 10000
NPAD = 10240
E = 640000
K = 160
EPAD = 32 * K * 128
F32 = jnp.float32
I32 = jnp.int32


def _tc_a_body(x_ref, w1p_ref, c1_ref, as8_ref, hx_ref, att_ref):
    hx = jnp.dot(x_ref[...], w1p_ref[...], preferred_element_type=F32) + c1_ref[...]
    hx_ref[...] = hx
    a8 = jnp.dot(hx, as8_ref[...], preferred_element_type=F32)
    rid = lax.broadcasted_iota(I32, (NPAD, 8), 0)
    att_ref[...] = jnp.where(rid < N, a8, -1e30)


def _tc_b_body(p0_ref, p1_ref, rp1_ref, b1_ref, w2p_ref, c2_ref, as2_ref,
               hx_ref, att_ref):
    acc = p0_ref[...] + p1_ref[...]
    num = acc[:, :32]
    denb = jnp.dot(acc, rp1_ref[...], preferred_element_type=F32)
    out1 = num / (denb + 1e-16) + b1_ref[...]
    x2 = jnp.where(out1 > 0, out1, jnp.exp(out1) - 1.0)
    hx2 = jnp.dot(x2, w2p_ref[...], preferred_element_type=F32) + c2_ref[...]
    hx_ref[...] = hx2
    a2 = jnp.dot(hx2, as2_ref[...], preferred_element_type=F32)
    rid = lax.broadcasted_iota(I32, (NPAD, 2), 0)
    att_ref[...] = jnp.where(rid < N, a2, -1e30)


def _tc_c_body(p0_ref, p1_ref, rp2_ref, b2_ref, out_ref):
    acc = p0_ref[...] + p1_ref[...]
    num = acc[:, :64]
    denb = jnp.dot(acc, rp2_ref[...], preferred_element_type=F32)
    out_ref[...] = num / (denb + 1e-16) + b2_ref[...]


_tc_a = pl.pallas_call(
    _tc_a_body,
    out_shape=[jax.ShapeDtypeStruct((NPAD, 48), F32),
               jax.ShapeDtypeStruct((NPAD, 8), F32)],
)

_tc_b = pl.pallas_call(
    _tc_b_body,
    out_shape=[jax.ShapeDtypeStruct((NPAD, 80), F32),
               jax.ShapeDtypeStruct((NPAD, 2), F32)],
)

_tc_c = pl.pallas_call(
    _tc_c_body,
    out_shape=jax.ShapeDtypeStruct((NPAD, 64), F32),
)


def _make_sc(rw, heads):
    ac = 2 * heads
    nv = rw // 16

    mesh = plsc.VectorSubcoreMesh(core_axis_name="c", subcore_axis_name="s",
                                  num_cores=2, num_subcores=16)

    @functools.partial(
        pl.kernel,
        out_type=jax.ShapeDtypeStruct((2, NPAD, rw), F32),
        mesh=mesh,
        compiler_params=pltpu.CompilerParams(needs_layout_passes=False,
                                             use_tc_tiling_on_sc=False),
        scratch_types=[
            pltpu.VMEM((NPAD * ac,), F32),
            pltpu.VMEM((2, 2, 128), I32),
            pltpu.VMEM((2, 128, rw), F32),
            pltpu.VMEM(((heads + 1) * 128,), F32),
            pltpu.VMEM((64, rw), F32),
            pltpu.SemaphoreType.DMA,
            pltpu.SemaphoreType.DMA,
            pltpu.VMEM_SHARED((NPAD, rw), F32),
        ],
    )
    def sc_kernel(idx_hbm, hx_hbm, att_hbm, out_hbm,
                  att_v, ibuf, hbuf, wbuf, zbuf, sem_g, sem_s, accum):
        cid = lax.axis_index("c")
        sid = lax.axis_index("s")
        wid = sid * 2 + cid
        base = sid * (NPAD // 16)

        def zr(r, _):
            for cc in range(nv):
                zbuf[r, pl.ds(cc * 16, 16)] = jnp.zeros((16,), F32)
            return 0
        lax.fori_loop(0, 64, zr, 0)

        def zc(k, _):
            pltpu.sync_copy(zbuf, accum.at[pl.ds(base + k * 64, 64)])
            return 0
        lax.fori_loop(0, NPAD // 16 // 64, zc, 0)

        pltpu.sync_copy(att_hbm, att_v)
        for cc in range(8):
            wbuf[pl.ds(heads * 128 + cc * 16, 16)] = jnp.zeros((16,), F32)

        plsc.subcore_barrier()

        def _rowpats():
            iota16 = lax.iota(I32, 16)
            if heads == 4:
                return [(iota16 // 8) * 128, (iota16 // 8) * 128 + 256,
                        jnp.minimum(iota16, 4) * 128]
            pats = [iota16 * 0 for _ in range(nv - 1)]
            pats.append(jnp.minimum(iota16, 1) * 128)
            return pats

        pltpu.sync_copy(idx_hbm.at[wid, 0], ibuf.at[0])
        pltpu.async_copy(hx_hbm.at[ibuf.at[0, 0]], hbuf.at[0], sem_g)

        def halfstep(j, slot):
            nxt = 1 - slot

            @pl.when(j > 0)
            def _wait_prev_scatter():
                pltpu.make_async_copy(
                    hbuf.at[nxt], accum.at[ibuf.at[nxt, 1]], sem_s).wait()

            @pl.when(j + 1 < K)
            def _prefetch_next():
                pltpu.sync_copy(idx_hbm.at[wid, j + 1], ibuf.at[nxt])
                pltpu.async_copy(hx_hbm.at[ibuf.at[nxt, 0]], hbuf.at[nxt], sem_g)

            pltpu.make_async_copy(
                hx_hbm.at[ibuf.at[slot, 0]], hbuf.at[slot], sem_g).wait()

            for g in range(8):
                s16 = ibuf[slot, 0, pl.ds(g * 16, 16)]
                d16 = ibuf[slot, 1, pl.ds(g * 16, 16)]
                for h in range(heads):
                    zs = plsc.load_gather(att_v, [s16 * ac + h])
                    zd = plsc.load_gather(att_v, [d16 * ac + (heads + h)])
                    z = zs + zd
                    wbuf[pl.ds(h * 128 + g * 16, 16)] = jnp.exp(jnp.maximum(z, 0.2 * z))

            def pedge(e2, _):
                rowpats = _rowpats()
                for ee in range(2):
                    e = e2 * 2 + ee
                    for v in range(nv):
                        wv = plsc.load_gather(wbuf, [rowpats[v] + e])
                        hbuf[slot, e, pl.ds(v * 16, 16)] = (
                            hbuf[slot, e, pl.ds(v * 16, 16)] * wv)
                return 0
            lax.fori_loop(0, 64, pedge, 0)

            pltpu.async_copy(hbuf.at[slot], accum.at[ibuf.at[slot, 1]], sem_s,
                             add=True)

        def body(i, _):
            halfstep(i * 2, 0)
            halfstep(i * 2 + 1, 1)
            return 0
        lax.fori_loop(0, K // 2, body, 0)
        pltpu.make_async_copy(hbuf.at[1], accum.at[ibuf.at[1, 1]], sem_s).wait()

        plsc.subcore_barrier()

        def co(k, _):
            pltpu.sync_copy(accum.at[pl.ds(base + k * 64, 64)], zbuf)
            pltpu.sync_copy(zbuf, out_hbm.at[cid, pl.ds(base + k * 64, 64)])
            return 0
        lax.fori_loop(0, NPAD // 16 // 64, co, 0)

    return sc_kernel


_sc1 = _make_sc(48, 4)
_sc2 = _make_sc(80, 1)


def kernel(x, edge_index, W1, att_src1, att_dst1, b1, W2, att_src2, att_dst2, b2):
    src = edge_index[0].astype(I32)
    dst = edge_index[1].astype(I32)
    loop = jnp.arange(N, dtype=I32)
    padn = EPAD - (E + N)
    src_p = jnp.concatenate([src, loop, jnp.zeros((padn,), I32)])
    dst_p = jnp.concatenate([dst, loop, N + (jnp.arange(padn, dtype=I32) % 240)])
    idx = jnp.stack([src_p, dst_p]).reshape(2, 32, K, 128).transpose(1, 2, 0, 3)

    x_pad = jnp.pad(x, ((0, NPAD - N), (0, 0)))

    m4 = jnp.repeat(jnp.eye(4, dtype=F32), 8, axis=0)
    as8 = jnp.concatenate([m4 * att_src1.reshape(32, 1),
                           m4 * att_dst1.reshape(32, 1)], axis=1)
    as8p = jnp.concatenate([as8, jnp.zeros((16, 8), F32)], axis=0)
    w1p = jnp.concatenate([W1, jnp.zeros((128, 16), F32)], axis=1)
    c1 = jnp.concatenate([jnp.zeros((32,), F32), jnp.ones((4,), F32),
                          jnp.zeros((12,), F32)]).reshape(1, 48)
    rp1 = jnp.zeros((48, 32), F32).at[32:36].set(m4.T)
    w2p = jnp.concatenate([W2, jnp.zeros((32, 16), F32)], axis=1)
    c2 = jnp.zeros((80,), F32).at[64].set(1.0).reshape(1, 80)
    as2p = jnp.zeros((80, 2), F32).at[:64].set(
        jnp.concatenate([att_src2.reshape(64, 1), att_dst2.reshape(64, 1)], axis=1))
    rp2 = jnp.zeros((80, 64), F32).at[64].set(jnp.ones((64,), F32))
    b1r = b1.reshape(1, 32)
    b2r = b2.reshape(1, 64)

    hx1, att1 = _tc_a(x_pad, w1p, c1, as8p)
    parts1 = _sc1(idx, hx1, att1.reshape(-1))
    hx2, att2 = _tc_b(parts1[0], parts1[1], rp1, b1r, w2p, c2, as2p)
    parts2 = _sc2(idx, hx2, att2.reshape(-1))
    out = _tc_c(parts2[0], parts2[1], rp2, b2r)
    return out[:N]


# speedup vs baseline: 79.1706x; 1.5622x over previous
import functools

import jax
import jax.numpy as jnp
from jax import lax
from jax.experimental import pallas as pl
from jax.experimental.pallas import tpu as pltpu
from jax.experimental.pallas import tpu_sc as plsc

N = 10000
NPAD = 10240
E = 640000
K = 160
EPAD = 32 * K * 128
F32 = jnp.float32
I32 = jnp.int32


def _tc_a_body(x_ref, w1p_ref, c1_ref, as8_ref, hx_ref, att_ref):
    hx = jnp.dot(x_ref[...], w1p_ref[...], preferred_element_type=F32) + c1_ref[...]
    hx_ref[...] = hx
    a8 = jnp.dot(hx, as8_ref[...], preferred_element_type=F32)
    rid = lax.broadcasted_iota(I32, (NPAD, 8), 0)
    att_ref[...] = jnp.where(rid < N, a8, -1e30)


def _tc_b_body(p0_ref, p1_ref, rp1_ref, b1_ref, w2p_ref, c2_ref, as2_ref,
               hx_ref, att_ref):
    acc = p0_ref[...] + p1_ref[...]
    num = acc[:, :32]
    denb = jnp.dot(acc, rp1_ref[...], preferred_element_type=F32)
    out1 = num / (denb + 1e-16) + b1_ref[...]
    x2 = jnp.where(out1 > 0, out1, jnp.exp(out1) - 1.0)
    hx2 = jnp.dot(x2, w2p_ref[...], preferred_element_type=F32) + c2_ref[...]
    hx_ref[...] = hx2
    a2 = jnp.dot(hx2, as2_ref[...], preferred_element_type=F32)
    rid = lax.broadcasted_iota(I32, (NPAD, 2), 0)
    att_ref[...] = jnp.where(rid < N, a2, -1e30)


def _tc_c_body(p0_ref, p1_ref, rp2_ref, b2_ref, out_ref):
    acc = p0_ref[...] + p1_ref[...]
    num = acc[:, :64]
    denb = jnp.dot(acc, rp2_ref[...], preferred_element_type=F32)
    out_ref[...] = num / (denb + 1e-16) + b2_ref[...]


_tc_a = pl.pallas_call(
    _tc_a_body,
    out_shape=[jax.ShapeDtypeStruct((NPAD, 48), F32),
               jax.ShapeDtypeStruct((NPAD, 8), F32)],
)

_tc_b = pl.pallas_call(
    _tc_b_body,
    out_shape=[jax.ShapeDtypeStruct((NPAD, 80), F32),
               jax.ShapeDtypeStruct((NPAD, 2), F32)],
)

_tc_c = pl.pallas_call(
    _tc_c_body,
    out_shape=jax.ShapeDtypeStruct((NPAD, 64), F32),
)


def _make_sc(rw, heads):
    ac = 2 * heads
    nv = rw // 16

    mesh = plsc.VectorSubcoreMesh(core_axis_name="c", subcore_axis_name="s",
                                  num_cores=2, num_subcores=16)

    @functools.partial(
        pl.kernel,
        out_type=jax.ShapeDtypeStruct((2, NPAD, rw), F32),
        mesh=mesh,
        compiler_params=pltpu.CompilerParams(needs_layout_passes=False,
                                             use_tc_tiling_on_sc=False),
        scratch_types=[
            pltpu.VMEM((NPAD * ac,), F32),
            pltpu.VMEM((2, 2, 128), I32),
            pltpu.VMEM((2, 128, rw), F32),
            pltpu.VMEM(((heads + 1) * 128,), F32),
            pltpu.VMEM((64, rw), F32),
            pltpu.SemaphoreType.DMA,
            pltpu.SemaphoreType.DMA,
            pltpu.VMEM_SHARED((NPAD, rw), F32),
        ],
    )
    def sc_kernel(idx_hbm, hx_hbm, att_hbm, out_hbm,
                  att_v, ibuf, hbuf, wbuf, zbuf, sem_g, sem_s, accum):
        cid = lax.axis_index("c")
        sid = lax.axis_index("s")
        wid = sid * 2 + cid
        base = sid * (NPAD // 16)

        def zr(r, _):
            for cc in range(nv):
                zbuf[r, pl.ds(cc * 16, 16)] = jnp.zeros((16,), F32)
            return 0
        lax.fori_loop(0, 64, zr, 0)

        def zc(k, _):
            pltpu.sync_copy(zbuf, accum.at[pl.ds(base + k * 64, 64)])
            return 0
        lax.fori_loop(0, NPAD // 16 // 64, zc, 0)

        pltpu.sync_copy(att_hbm, att_v)
        for cc in range(8):
            wbuf[pl.ds(heads * 128 + cc * 16, 16)] = jnp.zeros((16,), F32)

        plsc.subcore_barrier()

        def _rowpats():
            iota16 = lax.iota(I32, 16)
            if heads == 4:
                return [(iota16 // 8) * 128, (iota16 // 8) * 128 + 256,
                        jnp.minimum(iota16, 4) * 128]
            pats = [iota16 * 0 for _ in range(nv - 1)]
            pats.append(jnp.minimum(iota16, 1) * 128)
            return pats

        pltpu.sync_copy(idx_hbm.at[wid, 0], ibuf.at[0])
        pltpu.async_copy(hx_hbm.at[ibuf.at[0, 0]], hbuf.at[0], sem_g)

        def halfstep(j, slot):
            nxt = 1 - slot

            @pl.when(j > 0)
            def _wait_prev_scatter():
                pltpu.make_async_copy(
                    hbuf.at[nxt], accum.at[ibuf.at[nxt, 1]], sem_s).wait()

            @pl.when(j + 1 < K)
            def _prefetch_next():
                pltpu.sync_copy(idx_hbm.at[wid, j + 1], ibuf.at[nxt])
                pltpu.async_copy(hx_hbm.at[ibuf.at[nxt, 0]], hbuf.at[nxt], sem_g)

            pltpu.make_async_copy(
                hx_hbm.at[ibuf.at[slot, 0]], hbuf.at[slot], sem_g).wait()

            for g in range(8):
                s16 = ibuf[slot, 0, pl.ds(g * 16, 16)]
                d16 = ibuf[slot, 1, pl.ds(g * 16, 16)]
                for h in range(heads):
                    zs = plsc.load_gather(att_v, [s16 * ac + h])
                    zd = plsc.load_gather(att_v, [d16 * ac + (heads + h)])
                    z = zs + zd
                    wbuf[pl.ds(h * 128 + g * 16, 16)] = jnp.exp(jnp.maximum(z, 0.2 * z))

            @plsc.parallel_loop(0, 128, step=1, unroll=8,
                                carry=tuple(_rowpats()))
            def pedge(e, pats):
                for v in range(nv):
                    wv = plsc.load_gather(wbuf, [pats[v] + e])
                    hbuf[slot, e, pl.ds(v * 16, 16)] = (
                        hbuf[slot, e, pl.ds(v * 16, 16)] * wv)
                return pats

            pltpu.async_copy(hbuf.at[slot], accum.at[ibuf.at[slot, 1]], sem_s,
                             add=True)

        def body(i, _):
            halfstep(i * 2, 0)
            halfstep(i * 2 + 1, 1)
            return 0
        lax.fori_loop(0, K // 2, body, 0)
        pltpu.make_async_copy(hbuf.at[1], accum.at[ibuf.at[1, 1]], sem_s).wait()

        plsc.subcore_barrier()

        def co(k, _):
            pltpu.sync_copy(accum.at[pl.ds(base + k * 64, 64)], zbuf)
            pltpu.sync_copy(zbuf, out_hbm.at[cid, pl.ds(base + k * 64, 64)])
            return 0
        lax.fori_loop(0, NPAD // 16 // 64, co, 0)

    return sc_kernel


_sc1 = _make_sc(48, 4)
_sc2 = _make_sc(80, 1)


def kernel(x, edge_index, W1, att_src1, att_dst1, b1, W2, att_src2, att_dst2, b2):
    src = edge_index[0].astype(I32)
    dst = edge_index[1].astype(I32)
    loop = jnp.arange(N, dtype=I32)
    padn = EPAD - (E + N)
    src_p = jnp.concatenate([src, loop, jnp.zeros((padn,), I32)])
    dst_p = jnp.concatenate([dst, loop, N + (jnp.arange(padn, dtype=I32) % 240)])
    idx = jnp.stack([src_p, dst_p]).reshape(2, 32, K, 128).transpose(1, 2, 0, 3)

    x_pad = jnp.pad(x, ((0, NPAD - N), (0, 0)))

    m4 = jnp.repeat(jnp.eye(4, dtype=F32), 8, axis=0)
    as8 = jnp.concatenate([m4 * att_src1.reshape(32, 1),
                           m4 * att_dst1.reshape(32, 1)], axis=1)
    as8p = jnp.concatenate([as8, jnp.zeros((16, 8), F32)], axis=0)
    w1p = jnp.concatenate([W1, jnp.zeros((128, 16), F32)], axis=1)
    c1 = jnp.concatenate([jnp.zeros((32,), F32), jnp.ones((4,), F32),
                          jnp.zeros((12,), F32)]).reshape(1, 48)
    rp1 = jnp.zeros((48, 32), F32).at[32:36].set(m4.T)
    w2p = jnp.concatenate([W2, jnp.zeros((32, 16), F32)], axis=1)
    c2 = jnp.zeros((80,), F32).at[64].set(1.0).reshape(1, 80)
    as2p = jnp.zeros((80, 2), F32).at[:64].set(
        jnp.concatenate([att_src2.reshape(64, 1), att_dst2.reshape(64, 1)], axis=1))
    rp2 = jnp.zeros((80, 64), F32).at[64].set(jnp.ones((64,), F32))
    b1r = b1.reshape(1, 32)
    b2r = b2.reshape(1, 64)

    hx1, att1 = _tc_a(x_pad, w1p, c1, as8p)
    parts1 = _sc1(idx, hx1, att1.reshape(-1))
    hx2, att2 = _tc_b(parts1[0], parts1[1], rp1, b1r, w2p, c2, as2p)
    parts2 = _sc2(idx, hx2, att2.reshape(-1))
    out = _tc_c(parts2[0], parts2[1], rp2, b2r)
    return out[:N]


# speedup vs baseline: 79.7843x; 1.0078x over previous
import functools

import jax
import jax.numpy as jnp
from jax import lax
from jax.experimental import pallas as pl
from jax.experimental.pallas import tpu as pltpu
from jax.experimental.pallas import tpu_sc as plsc

N = 10000
NPAD = 10240
E = 640000
K = 160
EPAD = 32 * K * 128
F32 = jnp.float32
I32 = jnp.int32


def _tc_a_body(x_ref, w1p_ref, c1_ref, as8_ref, hx_ref, att_ref):
    hx = jnp.dot(x_ref[...], w1p_ref[...], preferred_element_type=F32) + c1_ref[...]
    hx_ref[...] = hx
    a8 = jnp.dot(hx, as8_ref[...], preferred_element_type=F32)
    rid = lax.broadcasted_iota(I32, (NPAD, 8), 0)
    att_ref[...] = jnp.where(rid < N, a8, -1e30)


def _tc_b_body(p0_ref, p1_ref, rp1_ref, b1_ref, w2p_ref, c2_ref, as2_ref,
               hx_ref, att_ref):
    acc = p0_ref[...] + p1_ref[...]
    num = acc[:, :32]
    denb = jnp.dot(acc, rp1_ref[...], preferred_element_type=F32)
    out1 = num / (denb + 1e-16) + b1_ref[...]
    x2 = jnp.where(out1 > 0, out1, jnp.exp(out1) - 1.0)
    hx2 = jnp.dot(x2, w2p_ref[...], preferred_element_type=F32) + c2_ref[...]
    hx_ref[...] = hx2
    a2 = jnp.dot(hx2, as2_ref[...], preferred_element_type=F32)
    rid = lax.broadcasted_iota(I32, (NPAD, 2), 0)
    att_ref[...] = jnp.where(rid < N, a2, -1e30)


def _tc_c_body(p0_ref, p1_ref, rp2_ref, b2_ref, out_ref):
    acc = p0_ref[...] + p1_ref[...]
    num = acc[:, :64]
    denb = jnp.dot(acc, rp2_ref[...], preferred_element_type=F32)
    out_ref[...] = num / (denb + 1e-16) + b2_ref[...]


_tc_a = pl.pallas_call(
    _tc_a_body,
    out_shape=[jax.ShapeDtypeStruct((NPAD, 48), F32),
               jax.ShapeDtypeStruct((NPAD, 8), F32)],
)

_tc_b = pl.pallas_call(
    _tc_b_body,
    out_shape=[jax.ShapeDtypeStruct((NPAD, 80), F32),
               jax.ShapeDtypeStruct((NPAD, 2), F32)],
)

_tc_c = pl.pallas_call(
    _tc_c_body,
    out_shape=jax.ShapeDtypeStruct((NPAD, 64), F32),
)


def _make_sc(rw, heads):
    ac = 2 * heads
    nv = rw // 16

    mesh = plsc.VectorSubcoreMesh(core_axis_name="c", subcore_axis_name="s",
                                  num_cores=2, num_subcores=16)

    @functools.partial(
        pl.kernel,
        out_type=jax.ShapeDtypeStruct((2, NPAD, rw), F32),
        mesh=mesh,
        compiler_params=pltpu.CompilerParams(needs_layout_passes=False,
                                             use_tc_tiling_on_sc=False),
        scratch_types=[
            pltpu.VMEM((NPAD * ac,), F32),
            pltpu.VMEM((2, 2, 128), I32),
            pltpu.VMEM((2, 128, rw), F32),
            pltpu.VMEM(((heads + 1) * 128,), F32),
            pltpu.VMEM((64, rw), F32),
            pltpu.SemaphoreType.DMA,
            pltpu.SemaphoreType.DMA,
            pltpu.VMEM_SHARED((NPAD, rw), F32),
        ],
    )
    def sc_kernel(idx_hbm, hx_hbm, att_hbm, out_hbm,
                  att_v, ibuf, hbuf, wbuf, zbuf, sem_g, sem_s, accum):
        cid = lax.axis_index("c")
        sid = lax.axis_index("s")
        wid = sid * 2 + cid
        base = sid * (NPAD // 16)

        def zr(r, _):
            for cc in range(nv):
                zbuf[r, pl.ds(cc * 16, 16)] = jnp.zeros((16,), F32)
            return 0
        lax.fori_loop(0, 64, zr, 0)

        def zc(k, _):
            pltpu.sync_copy(zbuf, accum.at[pl.ds(base + k * 64, 64)])
            return 0
        lax.fori_loop(0, NPAD // 16 // 64, zc, 0)

        pltpu.sync_copy(att_hbm, att_v)
        for cc in range(8):
            wbuf[pl.ds(heads * 128 + cc * 16, 16)] = jnp.zeros((16,), F32)

        plsc.subcore_barrier()

        def _rowpats():
            iota16 = lax.iota(I32, 16)
            if heads == 4:
                return [(iota16 // 8) * 128, (iota16 // 8) * 128 + 256,
                        jnp.minimum(iota16, 4) * 128]
            pats = [iota16 * 0 for _ in range(nv - 1)]
            pats.append(jnp.minimum(iota16, 1) * 128)
            return pats

        pltpu.sync_copy(idx_hbm.at[wid, 0], ibuf.at[0])
        pltpu.async_copy(hx_hbm.at[ibuf.at[0, 0]], hbuf.at[0], sem_g)

        def halfstep(j, slot):
            nxt = 1 - slot

            @pl.when(j > 0)
            def _wait_prev_scatter():
                pltpu.make_async_copy(
                    hbuf.at[nxt], accum.at[ibuf.at[nxt, 1]], sem_s).wait()

            @pl.when(j + 1 < K)
            def _prefetch_next():
                pltpu.sync_copy(idx_hbm.at[wid, j + 1], ibuf.at[nxt])
                pltpu.async_copy(hx_hbm.at[ibuf.at[nxt, 0]], hbuf.at[nxt], sem_g)

            pltpu.make_async_copy(
                hx_hbm.at[ibuf.at[slot, 0]], hbuf.at[slot], sem_g).wait()

            for g in range(8):
                s16 = ibuf[slot, 0, pl.ds(g * 16, 16)]
                d16 = ibuf[slot, 1, pl.ds(g * 16, 16)]
                for h in range(heads):
                    zs = plsc.load_gather(att_v, [s16 * ac + h])
                    zd = plsc.load_gather(att_v, [d16 * ac + (heads + h)])
                    z = zs + zd
                    wbuf[pl.ds(h * 128 + g * 16, 16)] = jnp.exp(jnp.maximum(z, 0.2 * z))

            @plsc.parallel_loop(0, 128, step=1, unroll=16,
                                carry=tuple(_rowpats()))
            def pedge(e, pats):
                for v in range(nv):
                    wv = plsc.load_gather(wbuf, [pats[v] + e])
                    hbuf[slot, e, pl.ds(v * 16, 16)] = (
                        hbuf[slot, e, pl.ds(v * 16, 16)] * wv)
                return pats

            pltpu.async_copy(hbuf.at[slot], accum.at[ibuf.at[slot, 1]], sem_s,
                             add=True)

        def body(i, _):
            halfstep(i * 2, 0)
            halfstep(i * 2 + 1, 1)
            return 0
        lax.fori_loop(0, K // 2, body, 0)
        pltpu.make_async_copy(hbuf.at[1], accum.at[ibuf.at[1, 1]], sem_s).wait()

        plsc.subcore_barrier()

        def co(k, _):
            pltpu.sync_copy(accum.at[pl.ds(base + k * 64, 64)], zbuf)
            pltpu.sync_copy(zbuf, out_hbm.at[cid, pl.ds(base + k * 64, 64)])
            return 0
        lax.fori_loop(0, NPAD // 16 // 64, co, 0)

    return sc_kernel


_sc1 = _make_sc(48, 4)
_sc2 = _make_sc(80, 1)


def kernel(x, edge_index, W1, att_src1, att_dst1, b1, W2, att_src2, att_dst2, b2):
    src = edge_index[0].astype(I32)
    dst = edge_index[1].astype(I32)
    loop = jnp.arange(N, dtype=I32)
    padn = EPAD - (E + N)
    src_p = jnp.concatenate([src, loop, jnp.zeros((padn,), I32)])
    dst_p = jnp.concatenate([dst, loop, N + (jnp.arange(padn, dtype=I32) % 240)])
    idx = jnp.stack([src_p, dst_p]).reshape(2, 32, K, 128).transpose(1, 2, 0, 3)

    x_pad = jnp.pad(x, ((0, NPAD - N), (0, 0)))

    m4 = jnp.repeat(jnp.eye(4, dtype=F32), 8, axis=0)
    as8 = jnp.concatenate([m4 * att_src1.reshape(32, 1),
                           m4 * att_dst1.reshape(32, 1)], axis=1)
    as8p = jnp.concatenate([as8, jnp.zeros((16, 8), F32)], axis=0)
    w1p = jnp.concatenate([W1, jnp.zeros((128, 16), F32)], axis=1)
    c1 = jnp.concatenate([jnp.zeros((32,), F32), jnp.ones((4,), F32),
                          jnp.zeros((12,), F32)]).reshape(1, 48)
    rp1 = jnp.zeros((48, 32), F32).at[32:36].set(m4.T)
    w2p = jnp.concatenate([W2, jnp.zeros((32, 16), F32)], axis=1)
    c2 = jnp.zeros((80,), F32).at[64].set(1.0).reshape(1, 80)
    as2p = jnp.zeros((80, 2), F32).at[:64].set(
        jnp.concatenate([att_src2.reshape(64, 1), att_dst2.reshape(64, 1)], axis=1))
    rp2 = jnp.zeros((80, 64), F32).at[64].set(jnp.ones((64,), F32))
    b1r = b1.reshape(1, 32)
    b2r = b2.reshape(1, 64)

    hx1, att1 = _tc_a(x_pad, w1p, c1, as8p)
    parts1 = _sc1(idx, hx1, att1.reshape(-1))
    hx2, att2 = _tc_b(parts1[0], parts1[1], rp1, b1r, w2p, c2, as2p)
    parts2 = _sc2(idx, hx2, att2.reshape(-1))
    out = _tc_c(parts2[0], parts2[1], rp2, b2r)
    return out[:N]
